# baseline (speedup 1.0000x reference)
import functools

import jax
import jax.numpy as jnp
from jax import lax
from jax.experimental import pallas as pl
from jax.experimental.pallas import tpu as pltpu
from jax.experimental.pallas import tpu_sc as plsc

N = 10000
E2 = 320000
HID = 512
CB = 8192


def _ttab_body(bond_ref, we_ref, be_ref, o_ref):
    o_ref[...] = jnp.dot(bond_ref[...], we_ref[...]) + be_ref[...]


def _edge_table(bond_emb, We, be):
    bond8 = jnp.pad(bond_emb, ((0, 3), (0, 0)))
    return pl.pallas_call(
        _ttab_body,
        in_specs=[
            pl.BlockSpec((8, 32), lambda: (0, 0)),
            pl.BlockSpec((32, HID), lambda: (0, 0)),
            pl.BlockSpec((HID,), lambda: (0,)),
        ],
        out_specs=pl.BlockSpec((8, HID), lambda: (0, 0)),
        out_shape=jax.ShapeDtypeStruct((8, HID), jnp.float32),
    )(bond8, We, be)


_CHUNK = 80
_PER_W = E2 // 32
_NSTEP = _PER_W // _CHUNK


def _sc_gather(Y, gidx):
    info = plsc.get_sparse_core_info()
    nc = info.num_cores

    @functools.partial(
        pl.kernel,
        mesh=plsc.VectorSubcoreMesh(core_axis_name="c", subcore_axis_name="s"),
        out_type=jax.ShapeDtypeStruct((E2, HID), jnp.float32),
        scratch_types=[
            pltpu.VMEM((_PER_W,), jnp.int32),
            pltpu.VMEM((_CHUNK, HID), jnp.float32),
            pltpu.VMEM((_CHUNK, HID), jnp.float32),
            pltpu.SemaphoreType.DMA,
            pltpu.SemaphoreType.DMA,
        ],
    )
    def k(y_hbm, gidx_hbm, out_hbm, idx_all, buf0, buf1, sem0, sem1):
        wid = lax.axis_index("s") * nc + lax.axis_index("c")
        base = wid * _PER_W
        pltpu.sync_copy(gidx_hbm.at[pl.ds(base, _PER_W)], idx_all)

        def halfstep(i, buf_v, sem, obuf):
            h = pltpu.async_copy(y_hbm.at[idx_all.at[pl.ds(i * _CHUNK, _CHUNK)]], buf_v, sem)

            @pl.when(i > 0)
            def _():
                pltpu.sync_copy(obuf, out_hbm.at[pl.ds(base + (i - 1) * _CHUNK, _CHUNK)])

            h.wait()

        def body(i, carry):
            halfstep(2 * i, buf0, sem0, buf1)
            halfstep(2 * i + 1, buf1, sem1, buf0)
            return carry

        lax.fori_loop(0, (_NSTEP - 1) // 2, body, 0)
        if _NSTEP % 2:
            halfstep(_NSTEP - 1, buf0, sem0, buf1)
            pltpu.sync_copy(buf0, out_hbm.at[pl.ds(base + (_NSTEP - 1) * _CHUNK, _CHUNK)])
        else:
            pltpu.sync_copy(buf1, out_hbm.at[pl.ds(base + (_NSTEP - 1) * _CHUNK, _CHUNK)])

    return k(Y, gidx)


def _wm1_body(h0_ref, h1_ref, h2_ref, h3_ref, w_ref, b_ref, o_ref):
    hs = [h0_ref, h1_ref, h2_ref, h3_ref]
    acc = jnp.dot(hs[3][...], w_ref[3 * HID:4 * HID, :])
    for j in (2, 1, 0):
        acc = acc + jnp.dot(hs[j][...], w_ref[j * HID:(j + 1) * HID, :])
    o_ref[...] = jnp.maximum(acc + b_ref[...], 0.0)


def _wm1(h0, h1, h2, h3, Wm1, bm1):
    return pl.pallas_call(
        _wm1_body,
        grid=(N // 400, 2),
        in_specs=[pl.BlockSpec((400, HID), lambda i, j: (i, 0)) for _ in range(4)] + [
            pl.BlockSpec((4 * HID, HID), lambda i, j: (0, j)),
            pl.BlockSpec((HID,), lambda i, j: (j,)),
        ],
        out_specs=pl.BlockSpec((400, HID), lambda i, j: (i, j)),
        out_shape=jax.ShapeDtypeStruct((N, 2 * HID), jnp.float32),
    )(h0, h1, h2, h3, Wm1, bm1)


def _vq_body(s_ref, h_ref, cb_ref, c2_ref, min_ref, idx_ref):
    h = h_ref[...]
    s = s_ref[...]
    for j in range(CB // 512):
        mm = jax.lax.dot_general(h, cb_ref[j * 512:(j + 1) * 512, :],
                                 (((1,), (1,)), ((), ())))
        d = s - 2.0 * mm + c2_ref[j * 512:(j + 1) * 512][None, :]
        rowmin = jnp.min(d, axis=1, keepdims=True)
        iota = jax.lax.broadcasted_iota(jnp.int32, d.shape, 1)
        rowidx = jnp.min(jnp.where(d == rowmin, iota, 2 ** 30), axis=1, keepdims=True)
        min_ref[:, j:j + 1] = rowmin
        idx_ref[:, j:j + 1] = rowidx + j * 512


def _vq_argmin(h_vq, cb):
    s = jnp.sum(h_vq * h_vq, axis=-1, keepdims=True)
    c2 = jnp.sum(cb * cb, axis=-1)
    nj = CB // 512
    mins, idxs = pl.pallas_call(
        _vq_body,
        grid=(N // 400,),
        in_specs=[
            pl.BlockSpec((400, 1), lambda i: (i, 0)),
            pl.BlockSpec((400, HID), lambda i: (i, 0)),
            pl.BlockSpec((CB, HID), lambda i: (0, 0)),
            pl.BlockSpec((CB,), lambda i: (0,)),
        ],
        out_specs=[
            pl.BlockSpec((400, nj), lambda i: (i, 0)),
            pl.BlockSpec((400, nj), lambda i: (i, 0)),
        ],
        out_shape=[
            jax.ShapeDtypeStruct((N, nj), jnp.float32),
            jax.ShapeDtypeStruct((N, nj), jnp.int32),
        ],
    )(s, h_vq, cb, c2)
    best = jnp.argmin(mins, axis=1)
    return jnp.take_along_axis(idxs, best[:, None], axis=1)[:, 0]


def _ln(x, g, b, eps=1e-5):
    m = jnp.mean(x, axis=-1, keepdims=True)
    v = jnp.var(x, axis=-1, keepdims=True)
    return (x - m) / jnp.sqrt(v + eps) * g + b


def _gine(x, src_unused, dst, gidx, lp, bond_emb):
    t = _edge_table(bond_emb, lp["We"], lp["be"])
    Y = jax.nn.relu(x[None, :, :] + t[:5, None, :]).reshape(5 * N, HID)
    m = _sc_gather(Y, gidx)
    aggr = jnp.zeros_like(x).at[dst].add(m)
    h = (1.0 + lp["eps"]) * x + aggr
    h = jax.nn.relu(h @ lp["W1"] + lp["b1"])
    h = jax.nn.relu(h @ lp["W2"] + lp["b2"])
    return h


def kernel(features, s1, d1, edge_weight, params):
    src = jnp.concatenate([s1, d1], axis=0)
    dst = jnp.concatenate([d1, s1], axis=0)
    e = jnp.concatenate([edge_weight, edge_weight], axis=0)
    e = jnp.where((e >= 1) & (e <= 4), e, jnp.zeros_like(e))
    gidx = (e * N + src).astype(jnp.int32)

    h0 = _ln(features @ params["W0"] + params["b0"], params["g_in"], params["bt_in"])
    l1, l2, l3 = params["layers"]
    h1 = _gine(h0, src, dst, gidx, l1, params["bond_emb"])
    h1 = _ln(h1 * l1["res"] + h0, l1["g"], l1["bt"])
    h2 = _gine(h1, src, dst, gidx, l2, params["bond_emb"])
    h2 = _ln(h2 * l2["res"] + h1, l2["g"], l2["bt"])
    h3 = _gine(h2, src, dst, gidx, l3, params["bond_emb"])
    h3 = _ln(h3 * l3["res"] + h2, l3["g"], l3["bt"])
    h_mid = _wm1(h0, h1, h2, h3, params["Wm1"], params["bm1"])
    h_mid = jax.nn.relu(h_mid @ params["Wm2"] + params["bm2"])
    h_out = h_mid @ params["Wo"] + params["bo"]
    h_vq = _ln(h_out, params["g_vq"], params["bt_vq"])
    cb = params["codebook"]
    idx = _vq_argmin(h_vq, cb)
    q = cb[idx]
    commit = jnp.mean((h_vq - jax.lax.stop_gradient(q)) ** 2)
    cb_loss = jnp.mean((jax.lax.stop_gradient(h_vq) - q) ** 2)
    quant = h_vq + jax.lax.stop_gradient(q - h_vq)
    loss = commit + cb_loss
    zero = jnp.zeros((), jnp.float32)
    return (loss, quant, commit, zero, zero)


# speedup vs baseline: 1.0072x; 1.0072x over previous
import functools

import jax
import jax.numpy as jnp
from jax import lax
from jax.experimental import pallas as pl
from jax.experimental.pallas import tpu as pltpu
from jax.experimental.pallas import tpu_sc as plsc

N = 10000
E2 = 320000
HID = 512
CB = 8192


def _ttab_body(bond_ref, we_ref, be_ref, o_ref):
    o_ref[...] = jnp.dot(bond_ref[...], we_ref[...]) + be_ref[...]


def _edge_table(bond_emb, We, be):
    bond8 = jnp.pad(bond_emb, ((0, 3), (0, 0)))
    return pl.pallas_call(
        _ttab_body,
        in_specs=[
            pl.BlockSpec((8, 32), lambda: (0, 0)),
            pl.BlockSpec((32, HID), lambda: (0, 0)),
            pl.BlockSpec((HID,), lambda: (0,)),
        ],
        out_specs=pl.BlockSpec((8, HID), lambda: (0, 0)),
        out_shape=jax.ShapeDtypeStruct((8, HID), jnp.float32),
    )(bond8, We, be)


_CHUNK = 80
_PER_W = E2 // 32
_NSTEP = _PER_W // _CHUNK


def _sc_gather(Y, gidx):
    info = plsc.get_sparse_core_info()
    nc = info.num_cores

    @functools.partial(
        pl.kernel,
        mesh=plsc.VectorSubcoreMesh(core_axis_name="c", subcore_axis_name="s"),
        out_type=jax.ShapeDtypeStruct((E2, HID), jnp.float32),
        scratch_types=[
            pltpu.VMEM((_PER_W,), jnp.int32),
            pltpu.VMEM((_CHUNK, HID), jnp.float32),
            pltpu.VMEM((_CHUNK, HID), jnp.float32),
            pltpu.SemaphoreType.DMA,
            pltpu.SemaphoreType.DMA,
        ],
    )
    def k(y_hbm, gidx_hbm, out_hbm, idx_all, buf0, buf1, sem0, sem1):
        wid = lax.axis_index("s") * nc + lax.axis_index("c")
        base = wid * _PER_W
        pltpu.sync_copy(gidx_hbm.at[pl.ds(base, _PER_W)], idx_all)

        def halfstep(i, buf_v, sem, obuf):
            h = pltpu.async_copy(y_hbm.at[idx_all.at[pl.ds(i * _CHUNK, _CHUNK)]], buf_v, sem)

            @pl.when(i > 0)
            def _():
                pltpu.sync_copy(obuf, out_hbm.at[pl.ds(base + (i - 1) * _CHUNK, _CHUNK)])

            h.wait()

        def body(i, carry):
            halfstep(2 * i, buf0, sem0, buf1)
            halfstep(2 * i + 1, buf1, sem1, buf0)
            return carry

        lax.fori_loop(0, (_NSTEP - 1) // 2, body, 0)
        if _NSTEP % 2:
            halfstep(_NSTEP - 1, buf0, sem0, buf1)
            pltpu.sync_copy(buf0, out_hbm.at[pl.ds(base + (_NSTEP - 1) * _CHUNK, _CHUNK)])
        else:
            pltpu.sync_copy(buf1, out_hbm.at[pl.ds(base + (_NSTEP - 1) * _CHUNK, _CHUNK)])

    return k(Y, gidx)


def _vq_body(s_ref, h_ref, cb_ref, c2_ref, min_ref, idx_ref):
    h = h_ref[...]
    s = s_ref[...]
    for j in range(CB // 512):
        mm = jax.lax.dot_general(h, cb_ref[j * 512:(j + 1) * 512, :],
                                 (((1,), (1,)), ((), ())))
        d = s - 2.0 * mm + c2_ref[j * 512:(j + 1) * 512][None, :]
        rowmin = jnp.min(d, axis=1, keepdims=True)
        iota = jax.lax.broadcasted_iota(jnp.int32, d.shape, 1)
        rowidx = jnp.min(jnp.where(d == rowmin, iota, 2 ** 30), axis=1, keepdims=True)
        min_ref[:, j:j + 1] = rowmin
        idx_ref[:, j:j + 1] = rowidx + j * 512


def _vq_argmin(h_vq, cb):
    s = jnp.sum(h_vq * h_vq, axis=-1, keepdims=True)
    c2 = jnp.sum(cb * cb, axis=-1)
    nj = CB // 512
    mins, idxs = pl.pallas_call(
        _vq_body,
        grid=(N // 400,),
        in_specs=[
            pl.BlockSpec((400, 1), lambda i: (i, 0)),
            pl.BlockSpec((400, HID), lambda i: (i, 0)),
            pl.BlockSpec((CB, HID), lambda i: (0, 0)),
            pl.BlockSpec((CB,), lambda i: (0,)),
        ],
        out_specs=[
            pl.BlockSpec((400, nj), lambda i: (i, 0)),
            pl.BlockSpec((400, nj), lambda i: (i, 0)),
        ],
        out_shape=[
            jax.ShapeDtypeStruct((N, nj), jnp.float32),
            jax.ShapeDtypeStruct((N, nj), jnp.int32),
        ],
    )(s, h_vq, cb, c2)
    best = jnp.argmin(mins, axis=1)
    return jnp.take_along_axis(idxs, best[:, None], axis=1)[:, 0]


def _ln(x, g, b, eps=1e-5):
    m = jnp.mean(x, axis=-1, keepdims=True)
    v = jnp.var(x, axis=-1, keepdims=True)
    return (x - m) / jnp.sqrt(v + eps) * g + b


def _gine(x, src_unused, dst, gidx, lp, bond_emb):
    t = _edge_table(bond_emb, lp["We"], lp["be"])
    Y = jax.nn.relu(x[None, :, :] + t[:5, None, :]).reshape(5 * N, HID)
    m = _sc_gather(Y, gidx)
    aggr = jnp.zeros_like(x).at[dst].add(m)
    h = (1.0 + lp["eps"]) * x + aggr
    h = jax.nn.relu(h @ lp["W1"] + lp["b1"])
    h = jax.nn.relu(h @ lp["W2"] + lp["b2"])
    return h


def kernel(features, s1, d1, edge_weight, params):
    src = jnp.concatenate([s1, d1], axis=0)
    dst = jnp.concatenate([d1, s1], axis=0)
    e = jnp.concatenate([edge_weight, edge_weight], axis=0)
    e = jnp.where((e >= 1) & (e <= 4), e, jnp.zeros_like(e))
    gidx = (e * N + src).astype(jnp.int32)

    h0 = _ln(features @ params["W0"] + params["b0"], params["g_in"], params["bt_in"])
    l1, l2, l3 = params["layers"]
    h1 = _gine(h0, src, dst, gidx, l1, params["bond_emb"])
    h1 = _ln(h1 * l1["res"] + h0, l1["g"], l1["bt"])
    h2 = _gine(h1, src, dst, gidx, l2, params["bond_emb"])
    h2 = _ln(h2 * l2["res"] + h1, l2["g"], l2["bt"])
    h3 = _gine(h2, src, dst, gidx, l3, params["bond_emb"])
    h3 = _ln(h3 * l3["res"] + h2, l3["g"], l3["bt"])
    h_cat = jnp.concatenate([h0, h1, h2, h3], axis=-1)
    h_mid = jax.nn.relu(h_cat @ params["Wm1"] + params["bm1"])
    h_mid = jax.nn.relu(h_mid @ params["Wm2"] + params["bm2"])
    h_out = h_mid @ params["Wo"] + params["bo"]
    h_vq = _ln(h_out, params["g_vq"], params["bt_vq"])
    cb = params["codebook"]
    idx = _vq_argmin(h_vq, cb)
    q = cb[idx]
    commit = jnp.mean((h_vq - jax.lax.stop_gradient(q)) ** 2)
    cb_loss = jnp.mean((jax.lax.stop_gradient(h_vq) - q) ** 2)
    quant = h_vq + jax.lax.stop_gradient(q - h_vq)
    loss = commit + cb_loss
    zero = jnp.zeros((), jnp.float32)
    return (loss, quant, commit, zero, zero)
